# baseline (speedup 1.0000x reference)
import functools

import jax
import jax.numpy as jnp
from jax import lax
from jax.experimental import pallas as pl
from jax.experimental.pallas import tpu as pltpu
from jax.experimental.pallas import tpu_sc as plsc
import dataclasses

from jax._src.pallas import mpmd
from jax._src.pallas import core as pallas_core
from jax._src.pallas.mosaic import core as tpu_core


def _sem_scratch(sem_type, shape, mesh):
    base = sem_type(shape)
    return dataclasses.replace(
        base,
        memory_space=pallas_core.CoreMemorySpace(
            tpu_core.MemorySpace.SEMAPHORE, mesh
        ),
    )

DIM = 4096
NUM_CORES = 2
NUM_SUBCORES = 16
NUM_WORKERS = NUM_CORES * NUM_SUBCORES
CHUNK = 4
NSLOT = 4
PAIR = 2 * CHUNK


@functools.partial(jax.jit, static_argnums=(2, 3))
def _gather_sc(idx, pe, b_per_w, n_chunks):
    vec_mesh = plsc.VectorSubcoreMesh(
        core_axis_name="c", subcore_axis_name="s", num_cores=NUM_CORES
    )
    scs_mesh = plsc.ScalarSubcoreMesh(axis_name="c", num_cores=NUM_CORES)
    n_rounds = n_chunks // 2
    assert n_chunks % NSLOT == 0 and n_rounds % 2 == 0

    def tec_fn(idx_hbm, table_hbm, out_hbm, ring, full, free, dmasems):
        def inner(idx_v, buf0, buf1, g0, g1, p0, p1):
            bufs = (buf0, buf1)
            gsem = (g0, g1)
            psem = (p0, p1)
            sid = lax.axis_index("s")
            wid = sid * NUM_CORES + lax.axis_index("c")
            pltpu.sync_copy(idx_hbm.at[wid], idx_v)

            def start_gather(j, b):
                pltpu.async_copy(table_hbm.at[idx_v.at[j]], bufs[b], gsem[b])

            def wait_gather(j, b):
                pltpu.make_async_copy(table_hbm.at[idx_v.at[j]], bufs[b], gsem[b]).wait()

            start_gather(0, 0)
            start_gather(1, 1)

            def body(i, carry):
                for kk in range(NSLOT):
                    j = i * NSLOT + kk
                    b = kk % 2
                    dst = ring.at[sid, pl.ds(kk * CHUNK, CHUNK)]

                    wait_gather(j, b)

                    @pl.when(i > 0)
                    def _():
                        pl.semaphore_wait(free.at[sid], 1)

                    pltpu.async_copy(bufs[b], dst, psem[b])
                    pltpu.make_async_copy(bufs[b], dst, psem[b]).wait()
                    pl.semaphore_signal(full.at[sid], 1)

                    if kk < 2:
                        start_gather(j + 2, b)
                    else:
                        @pl.when(i < n_chunks // NSLOT - 1)
                        def _():
                            start_gather(j + 2, b)
                return carry

            lax.fori_loop(0, n_chunks // NSLOT, body, 0)

        pl.run_scoped(
            inner,
            pltpu.VMEM((n_chunks, CHUNK), jnp.int32),
            pltpu.VMEM((CHUNK, DIM), jnp.float32),
            pltpu.VMEM((CHUNK, DIM), jnp.float32),
            pltpu.SemaphoreType.DMA,
            pltpu.SemaphoreType.DMA,
            pltpu.SemaphoreType.DMA,
            pltpu.SemaphoreType.DMA,
        )

    def scs_fn(idx_hbm, table_hbm, out_hbm, ring, full, free, dmasems):
        cid = lax.axis_index("c")

        def drain_src(s, p):
            return ring.at[s, pl.ds(p * PAIR, PAIR)]

        def body(k2, carry):
            for p in range(2):
                k = k2 * 2 + p
                for s in range(NUM_SUBCORES):
                    wid = s * NUM_CORES + cid
                    base = wid * b_per_w

                    @pl.when(k2 > 0)
                    def _():
                        pltpu.make_async_copy(
                            drain_src(s, p), out_hbm.at[pl.ds(base, PAIR)],
                            dmasems.at[s, p],
                        ).wait()
                        pl.semaphore_signal(free.at[s], 2, device_id={"s": s})

                    pl.semaphore_wait(full.at[s], 2)
                    pltpu.async_copy(
                        drain_src(s, p),
                        out_hbm.at[pl.ds(base + k * PAIR, PAIR)],
                        dmasems.at[s, p],
                    )
            return carry

        lax.fori_loop(0, n_rounds // 2, body, 0)

        for p in range(2):
            for s in range(NUM_SUBCORES):
                wid = s * NUM_CORES + cid
                base = wid * b_per_w
                pltpu.make_async_copy(
                    drain_src(s, p), out_hbm.at[pl.ds(base, PAIR)],
                    dmasems.at[s, p],
                ).wait()

    out = mpmd.mpmd_map(
        [(scs_mesh, scs_fn), (vec_mesh, tec_fn)],
        out_types=jax.ShapeDtypeStruct((NUM_WORKERS * b_per_w, DIM), jnp.float32),
        scratch_types=[
            pltpu.VMEM_SHARED((NUM_SUBCORES, NSLOT * CHUNK, DIM), jnp.float32),
            _sem_scratch(pltpu.SemaphoreType.REGULAR, (NUM_SUBCORES,), scs_mesh),
            _sem_scratch(pltpu.SemaphoreType.REGULAR, (NUM_SUBCORES,), vec_mesh),
            _sem_scratch(pltpu.SemaphoreType.DMA, (NUM_SUBCORES, 2), scs_mesh),
        ],
    )(idx, pe)
    return out


def kernel(t, pe):
    batch, seq = t.shape
    total = batch * seq
    b_per_w = total // NUM_WORKERS
    n_chunks = b_per_w // CHUNK
    idx = t.astype(jnp.int32).reshape(NUM_WORKERS, n_chunks, CHUNK)
    out = _gather_sc(idx, pe, b_per_w, n_chunks)
    return out.reshape(batch, seq, DIM)


# speedup vs baseline: 7.4706x; 7.4706x over previous
import functools

import jax
import jax.numpy as jnp
from jax import lax
from jax.experimental import pallas as pl
from jax.experimental.pallas import tpu as pltpu
from jax.experimental.pallas import tpu_sc as plsc

DIM = 4096
NUM_CORES = 2
NUM_SUBCORES = 16
NUM_WORKERS = NUM_CORES * NUM_SUBCORES
CHUNK = 8
NBUF = 3


@functools.partial(jax.jit, static_argnums=(2, 3))
def _gather_sc(idx, pe, b_per_w, n_chunks):
    mesh = plsc.VectorSubcoreMesh(
        core_axis_name="c", subcore_axis_name="s", num_cores=NUM_CORES
    )
    n_main = (n_chunks // NBUF) * NBUF if n_chunks % NBUF else n_chunks - NBUF
    while n_chunks - n_main < NBUF - 1:
        n_main -= NBUF

    @functools.partial(
        pl.kernel,
        out_type=jax.ShapeDtypeStruct((NUM_WORKERS * b_per_w, DIM), jnp.float32),
        mesh=mesh,
        compiler_params=pltpu.CompilerParams(use_tc_tiling_on_sc=True),
        scratch_types=[
            pltpu.VMEM((n_chunks, CHUNK), jnp.int32),
            *([pltpu.VMEM((CHUNK, DIM), jnp.float32)] * NBUF),
            *([pltpu.SemaphoreType.DMA] * (2 * NBUF)),
        ],
    )
    def k(idx_hbm, table_hbm, out_hbm, idx_v, *bufs_and_sems):
        bufs = bufs_and_sems[:NBUF]
        gsem = bufs_and_sems[NBUF : 2 * NBUF]
        ssem = bufs_and_sems[2 * NBUF :]

        wid = lax.axis_index("s") * NUM_CORES + lax.axis_index("c")
        base = wid * b_per_w

        pltpu.sync_copy(idx_hbm.at[wid], idx_v)

        def start_gather(j, b):
            pltpu.async_copy(table_hbm.at[idx_v.at[j]], bufs[b], gsem[b])

        def wait_gather(j, b):
            pltpu.make_async_copy(table_hbm.at[idx_v.at[j]], bufs[b], gsem[b]).wait()

        def start_scatter(j, b):
            pltpu.async_copy(bufs[b], out_hbm.at[pl.ds(base + j * CHUNK, CHUNK)], ssem[b])

        def wait_scatter(b):
            pltpu.make_async_copy(bufs[b], out_hbm.at[pl.ds(base, CHUNK)], ssem[b]).wait()

        start_gather(0, 0)
        start_gather(1, 1)

        def body(i, carry):
            j0 = i * NBUF
            for kk in range(NBUF):
                j = j0 + kk
                b = kk
                pf = (kk + 2) % NBUF
                wait_gather(j, b)
                start_scatter(j, b)
                if kk == 0:
                    @pl.when(i > 0)
                    def _():
                        wait_scatter(pf)
                else:
                    wait_scatter(pf)
                start_gather(j + 2, pf)
            return carry

        lax.fori_loop(0, n_main // NBUF, body, 0)

        for j in range(n_main, n_chunks):
            b = j % NBUF
            if j >= n_main + 2:
                wait_scatter(b)
                start_gather(j, b)
            wait_gather(j, b)
            start_scatter(j, b)

        for j in range(n_chunks - NBUF, n_chunks):
            wait_scatter(j % NBUF)

    return k(idx, pe)


def kernel(t, pe):
    batch, seq = t.shape
    total = batch * seq
    b_per_w = total // NUM_WORKERS
    n_chunks = b_per_w // CHUNK
    idx = t.astype(jnp.int32).reshape(NUM_WORKERS, n_chunks, CHUNK)
    out = _gather_sc(idx, pe, b_per_w, n_chunks)
    return out.reshape(batch, seq, DIM)
